# baseline (speedup 1.0000x reference)
import functools

import jax
import jax.numpy as jnp
from jax import lax
from jax.experimental import pallas as pl
from jax.experimental.pallas import tpu as pltpu
from jax.experimental.pallas import tpu_sc as plsc

NC, NS, L = 2, 16, 16

N_NODES = 10000
E_EDGES = 320000
DIM = 128
T_SEG = 512
R_SEG = 1024
H_DIM = 256

EDC = 128
EPT = 20480
NCHT = EPT // EDC
E_PAD = EPT * NS
ZR = 80
N_NCHUNK = N_NODES // ZR
POOL_C = 80
POOL_NCHUNK = N_NODES // POOL_C

_mesh = plsc.VectorSubcoreMesh(
    core_axis_name="c", subcore_axis_name="s", num_cores=NC, num_subcores=NS)


def _zero_vmem(buf, rows):
    def body(i, carry):
        r = i // (DIM // L)
        cc = (i % (DIM // L)) * L
        buf[r, pl.ds(cc, L)] = jnp.zeros((L,), jnp.float32)
        return carry
    lax.fori_loop(0, rows * (DIM // L), body, 0)


@functools.partial(
    pl.kernel,
    out_type=jax.ShapeDtypeStruct((2 * N_NODES, DIM), jnp.float32),
    mesh=_mesh,
    scratch_types=[
        pltpu.VMEM((EDC,), jnp.int32),
        pltpu.VMEM((EDC,), jnp.int32),
        pltpu.VMEM((EDC,), jnp.int32),
        pltpu.VMEM((EDC,), jnp.int32),
        pltpu.VMEM((EDC, DIM), jnp.float32),
        pltpu.VMEM((EDC, DIM), jnp.float32),
        pltpu.VMEM((ZR, DIM), jnp.float32),
        pltpu.VMEM_SHARED((N_NODES, DIM), jnp.float32),
        pltpu.SemaphoreType.DMA,
        pltpu.SemaphoreType.DMA,
        pltpu.SemaphoreType.DMA,
        pltpu.SemaphoreType.DMA,
    ],
)
def _edge_agg(src_all, dst_all, x_all, out, sbuf_a, sbuf_b, dbuf_a, dbuf_b,
              rows_a, rows_b, zbuf, acc, sem_a, sem_b, sem_ia, sem_ib):
    c = lax.axis_index("c")
    s = lax.axis_index("s")

    _zero_vmem(zbuf, ZR)

    def zbody(k, carry):
        g = s + k * NS

        @pl.when(g < N_NCHUNK)
        def _():
            pltpu.sync_copy(zbuf, acc.at[pl.ds(g * ZR, ZR)])
        return carry

    lax.fori_loop(0, (N_NCHUNK + NS - 1) // NS, zbody, 0)

    ebase = c * E_PAD + s * EPT

    pltpu.sync_copy(src_all.at[pl.ds(ebase, EDC)], sbuf_a)
    pltpu.sync_copy(dst_all.at[pl.ds(ebase, EDC)], dbuf_a)
    plsc.subcore_barrier()
    pltpu.async_copy(x_all.at[sbuf_a], rows_a, sem_a)
    pltpu.sync_copy(src_all.at[pl.ds(ebase + EDC, EDC)], sbuf_b)
    pltpu.sync_copy(dst_all.at[pl.ds(ebase + EDC, EDC)], dbuf_b)

    def body(k, carry):
        g0 = 2 * k
        more = g0 + 2 < NCHT

        pltpu.make_async_copy(x_all.at[sbuf_a], rows_a, sem_a).wait()
        pltpu.async_copy(x_all.at[sbuf_b], rows_b, sem_b)

        @pl.when(more)
        def _():
            base = ebase + (g0 + 2) * EDC
            pltpu.async_copy(src_all.at[pl.ds(base, EDC)], sbuf_a, sem_ia)

        pltpu.sync_copy(rows_a, acc.at[dbuf_a], add=True)

        @pl.when(more)
        def _():
            base = ebase + (g0 + 2) * EDC
            pltpu.async_copy(dst_all.at[pl.ds(base, EDC)], dbuf_a, sem_ia)

        pltpu.make_async_copy(x_all.at[sbuf_b], rows_b, sem_b).wait()

        @pl.when(more)
        def _():
            pltpu.make_async_copy(src_all.at[pl.ds(0, EDC)], sbuf_a, sem_ia).wait()
            pltpu.make_async_copy(dst_all.at[pl.ds(0, EDC)], dbuf_a, sem_ia).wait()
            pltpu.async_copy(x_all.at[sbuf_a], rows_a, sem_a)
            base = ebase + (g0 + 3) * EDC
            pltpu.async_copy(src_all.at[pl.ds(base, EDC)], sbuf_b, sem_ib)

        pltpu.sync_copy(rows_b, acc.at[dbuf_b], add=True)

        @pl.when(more)
        def _():
            base = ebase + (g0 + 3) * EDC
            pltpu.async_copy(dst_all.at[pl.ds(base, EDC)], dbuf_b, sem_ib)
            pltpu.make_async_copy(src_all.at[pl.ds(0, EDC)], sbuf_b, sem_ib).wait()
            pltpu.make_async_copy(dst_all.at[pl.ds(0, EDC)], dbuf_b, sem_ib).wait()
        return carry

    lax.fori_loop(0, NCHT // 2, body, 0)
    plsc.subcore_barrier()

    def obody(k, carry):
        g = s + k * NS

        @pl.when(g < N_NCHUNK)
        def _():
            pltpu.sync_copy(acc.at[pl.ds(g * ZR, ZR)],
                            out.at[pl.ds(c * N_NODES + g * ZR, ZR)])
        return carry

    lax.fori_loop(0, (N_NCHUNK + NS - 1) // NS, obody, 0)


_GNN_BLK = 1000


_NBLK = 2 * N_NODES // _GNN_BLK
_SEGS = T_SEG + R_SEG


def _fused_body(x_ref, a_ref, st_ref, en_ref, ridx_ref, wps, wpn, bp, wrs,
                wrn, br, w1, b1, w2, b2, o_ref, acc_ref):
    i = pl.program_id(0)
    is_prod = i < (N_NODES // _GNN_BLK)
    ws = jnp.where(is_prod, wps[...], wrs[...])
    wn = jnp.where(is_prod, wpn[...], wrn[...])
    b = jnp.where(is_prod, bp[...], br[...])
    h = jnp.maximum(
        jnp.dot(x_ref[...], ws, preferred_element_type=jnp.float32)
        + jnp.dot(a_ref[...], wn, preferred_element_type=jnp.float32) + b, 0.0)

    gr = lax.broadcasted_iota(jnp.int32, (_SEGS, _GNN_BLK), 1) + i * _GNN_BLK
    onehot = ((gr >= st_ref[...]) & (gr < en_ref[...])).astype(jnp.float32)
    part = jnp.dot(onehot, h, preferred_element_type=jnp.float32)

    @pl.when(i == 0)
    def _():
        acc_ref[...] = part

    @pl.when(i > 0)
    def _():
        acc_ref[...] += part

    @pl.when(i == _NBLK - 1)
    def _():
        prods = acc_ref[0:T_SEG, :]
        rxt = jnp.maximum(acc_ref[T_SEG:, :], 0.0)
        ids2 = ridx_ref[...]
        t2 = lax.broadcasted_iota(jnp.int32, (T_SEG, R_SEG), 0)
        oh2 = (t2 == ids2).astype(jnp.float32)
        pooled = jnp.dot(oh2, rxt, preferred_element_type=jnp.float32)
        feats = jnp.concatenate([prods, pooled], axis=1)
        h1 = jnp.maximum(
            jnp.dot(feats, w1[...], preferred_element_type=jnp.float32)
            + b1[...], 0.0)
        o_ref[...] = (jnp.dot(h1, w2[...], preferred_element_type=jnp.float32)
                      + b2[...])


def _fused_dense(x_all, agg, starts, ends, rx_idx, wps, wpn, bp, wrs, wrn, br,
                 w1, b1, w2, b2):
    wspec = pl.BlockSpec((DIM, DIM), lambda i: (0, 0))
    bspec = pl.BlockSpec((1, DIM), lambda i: (0, 0))
    return pl.pallas_call(
        _fused_body,
        grid=(_NBLK,),
        in_specs=[
            pl.BlockSpec((_GNN_BLK, DIM), lambda i: (i, 0)),
            pl.BlockSpec((_GNN_BLK, DIM), lambda i: (i, 0)),
            pl.BlockSpec((_SEGS, 1), lambda i: (0, 0)),
            pl.BlockSpec((_SEGS, 1), lambda i: (0, 0)),
            pl.BlockSpec((1, R_SEG), lambda i: (0, 0)),
            wspec, wspec, bspec, wspec, wspec, bspec,
            pl.BlockSpec((2 * DIM, H_DIM), lambda i: (0, 0)),
            pl.BlockSpec((1, H_DIM), lambda i: (0, 0)),
            pl.BlockSpec((H_DIM, DIM), lambda i: (0, 0)),
            bspec,
        ],
        out_specs=pl.BlockSpec((T_SEG, DIM), lambda i: (0, 0)),
        out_shape=jax.ShapeDtypeStruct((T_SEG, DIM), jnp.float32),
        scratch_shapes=[pltpu.VMEM((_SEGS, DIM), jnp.float32)],
    )(x_all, agg, starts, ends, rx_idx, wps, wpn, bp, wrs, wrn, br,
      w1, b1, w2, b2)


def kernel(x_prod, edge_index_prod, batch_prod, x_react, edge_index_react,
           batch_react, rxtant_indices, W_prod_self, W_prod_nbr, b_prod,
           W_react_self, W_react_nbr, b_react, W1, b1, W2, b2):
    x_all = jnp.concatenate(
        [x_prod, x_react, jnp.zeros((8, DIM), jnp.float32)], axis=0)
    npad = E_PAD - E_EDGES
    pad_iota = jnp.arange(npad, dtype=jnp.int32)
    src_fill = 2 * N_NODES + (pad_iota % 8)
    dst_fill = (pad_iota * 79) % N_NODES
    src_all = jnp.concatenate(
        [edge_index_prod[0], src_fill, edge_index_react[0] + N_NODES, src_fill])
    dst_all = jnp.concatenate(
        [edge_index_prod[1], dst_fill, edge_index_react[1], dst_fill])

    agg = _edge_agg(src_all, dst_all, x_all)

    bshift = jnp.concatenate([batch_prod, batch_react + T_SEG])
    starts = jnp.searchsorted(bshift, jnp.arange(_SEGS, dtype=bshift.dtype))
    starts = starts.astype(jnp.int32)
    ends = jnp.concatenate(
        [starts[1:], jnp.array([2 * N_NODES], jnp.int32)])
    return _fused_dense(x_all, agg, starts.reshape(_SEGS, 1),
                        ends.reshape(_SEGS, 1),
                        rxtant_indices.reshape(1, R_SEG),
                        W_prod_self, W_prod_nbr, b_prod.reshape(1, DIM),
                        W_react_self, W_react_nbr, b_react.reshape(1, DIM),
                        W1, b1.reshape(1, H_DIM), W2, b2.reshape(1, DIM))


# speedup vs baseline: 1.2736x; 1.2736x over previous
import functools

import jax
import jax.numpy as jnp
from jax import lax
from jax.experimental import pallas as pl
from jax.experimental.pallas import tpu as pltpu
from jax.experimental.pallas import tpu_sc as plsc

NC, NS, L = 2, 16, 16

N_NODES = 10000
E_EDGES = 320000
DIM = 128
T_SEG = 512
R_SEG = 1024
H_DIM = 256

EDC = 128
EPT = 20480
NCHT = EPT // EDC
E_PAD = EPT * NS
ZR = 80
N_NCHUNK = N_NODES // ZR
POOL_C = 80
POOL_NCHUNK = N_NODES // POOL_C

_mesh = plsc.VectorSubcoreMesh(
    core_axis_name="c", subcore_axis_name="s", num_cores=NC, num_subcores=NS)


def _zero_vmem(buf, rows):
    def body(i, carry):
        r = i // (DIM // L)
        cc = (i % (DIM // L)) * L
        buf[r, pl.ds(cc, L)] = jnp.zeros((L,), jnp.float32)
        return carry
    lax.fori_loop(0, rows * (DIM // L), body, 0)


@functools.partial(
    pl.kernel,
    out_type=jax.ShapeDtypeStruct((2 * N_NODES, DIM), jnp.float32),
    mesh=_mesh,
    scratch_types=[
        pltpu.VMEM((EDC,), jnp.int32),
        pltpu.VMEM((EDC,), jnp.int32),
        pltpu.VMEM((EDC,), jnp.int32),
        pltpu.VMEM((EDC,), jnp.int32),
        pltpu.VMEM((EDC, DIM), jnp.float32),
        pltpu.VMEM((EDC, DIM), jnp.float32),
        pltpu.VMEM((ZR, DIM), jnp.float32),
        pltpu.VMEM_SHARED((N_NODES, DIM), jnp.float32),
        pltpu.SemaphoreType.DMA,
        pltpu.SemaphoreType.DMA,
        pltpu.SemaphoreType.DMA,
        pltpu.SemaphoreType.DMA,
    ],
)
def _edge_agg(src_all, dst_all, x_all, out, sbuf_a, sbuf_b, dbuf_a, dbuf_b,
              rows_a, rows_b, zbuf, acc, sem_a, sem_b, sem_ia, sem_ib):
    c = lax.axis_index("c")
    s = lax.axis_index("s")

    _zero_vmem(zbuf, ZR)

    def zbody(k, carry):
        g = s + k * NS

        @pl.when(g < N_NCHUNK)
        def _():
            pltpu.sync_copy(zbuf, acc.at[pl.ds(g * ZR, ZR)])
        return carry

    lax.fori_loop(0, (N_NCHUNK + NS - 1) // NS, zbody, 0)

    ebase = c * E_PAD + s * EPT

    pltpu.sync_copy(src_all.at[pl.ds(ebase, EDC)], sbuf_a)
    pltpu.sync_copy(dst_all.at[pl.ds(ebase, EDC)], dbuf_a)
    plsc.subcore_barrier()
    pltpu.async_copy(x_all.at[sbuf_a], rows_a, sem_a)
    pltpu.sync_copy(src_all.at[pl.ds(ebase + EDC, EDC)], sbuf_b)
    pltpu.sync_copy(dst_all.at[pl.ds(ebase + EDC, EDC)], dbuf_b)

    def body(k, carry):
        g0 = 2 * k
        more = g0 + 2 < NCHT

        pltpu.make_async_copy(x_all.at[sbuf_a], rows_a, sem_a).wait()
        pltpu.async_copy(x_all.at[sbuf_b], rows_b, sem_b)

        @pl.when(more)
        def _():
            base = ebase + (g0 + 2) * EDC
            pltpu.async_copy(src_all.at[pl.ds(base, EDC)], sbuf_a, sem_ia)

        pltpu.sync_copy(rows_a, acc.at[dbuf_a], add=True)

        @pl.when(more)
        def _():
            base = ebase + (g0 + 2) * EDC
            pltpu.async_copy(dst_all.at[pl.ds(base, EDC)], dbuf_a, sem_ia)

        pltpu.make_async_copy(x_all.at[sbuf_b], rows_b, sem_b).wait()

        @pl.when(more)
        def _():
            pltpu.make_async_copy(src_all.at[pl.ds(0, EDC)], sbuf_a, sem_ia).wait()
            pltpu.make_async_copy(dst_all.at[pl.ds(0, EDC)], dbuf_a, sem_ia).wait()
            pltpu.async_copy(x_all.at[sbuf_a], rows_a, sem_a)
            base = ebase + (g0 + 3) * EDC
            pltpu.async_copy(src_all.at[pl.ds(base, EDC)], sbuf_b, sem_ib)

        pltpu.sync_copy(rows_b, acc.at[dbuf_b], add=True)

        @pl.when(more)
        def _():
            base = ebase + (g0 + 3) * EDC
            pltpu.async_copy(dst_all.at[pl.ds(base, EDC)], dbuf_b, sem_ib)
            pltpu.make_async_copy(src_all.at[pl.ds(0, EDC)], sbuf_b, sem_ib).wait()
            pltpu.make_async_copy(dst_all.at[pl.ds(0, EDC)], dbuf_b, sem_ib).wait()
        return carry

    lax.fori_loop(0, NCHT // 2, body, 0)
    plsc.subcore_barrier()

    def obody(k, carry):
        g = s + k * NS

        @pl.when(g < N_NCHUNK)
        def _():
            pltpu.sync_copy(acc.at[pl.ds(g * ZR, ZR)],
                            out.at[pl.ds(c * N_NODES + g * ZR, ZR)])
        return carry

    lax.fori_loop(0, (N_NCHUNK + NS - 1) // NS, obody, 0)


_GNN_BLK = 1000


_NBLK = 2 * N_NODES // _GNN_BLK
_SEGS = T_SEG + R_SEG


def _fused_body(x_ref, a_ref, bid_ref, ridx_ref, wps, wpn, bp, wrs,
                wrn, br, w1, b1, w2, b2, o_ref, acc_ref):
    i = pl.program_id(0)
    is_prod = i < (N_NODES // _GNN_BLK)
    ws = jnp.where(is_prod, wps[...], wrs[...])
    wn = jnp.where(is_prod, wpn[...], wrn[...])
    b = jnp.where(is_prod, bp[...], br[...])
    h = jnp.maximum(
        jnp.dot(x_ref[...], ws, preferred_element_type=jnp.float32)
        + jnp.dot(a_ref[...], wn, preferred_element_type=jnp.float32) + b, 0.0)

    seg = lax.broadcasted_iota(jnp.int32, (_GNN_BLK, _SEGS), 1)
    onehot = (seg == bid_ref[...]).astype(jnp.float32)
    part = lax.dot_general(onehot, h, (((0,), (0,)), ((), ())),
                           preferred_element_type=jnp.float32)

    @pl.when(i == 0)
    def _():
        acc_ref[...] = part

    @pl.when(i > 0)
    def _():
        acc_ref[...] += part

    @pl.when(i == _NBLK - 1)
    def _():
        prods = acc_ref[0:T_SEG, :]
        rxt = jnp.maximum(acc_ref[T_SEG:, :], 0.0)
        ids2 = ridx_ref[...]
        t2 = lax.broadcasted_iota(jnp.int32, (T_SEG, R_SEG), 0)
        oh2 = (t2 == ids2).astype(jnp.float32)
        pooled = jnp.dot(oh2, rxt, preferred_element_type=jnp.float32)
        feats = jnp.concatenate([prods, pooled], axis=1)
        h1 = jnp.maximum(
            jnp.dot(feats, w1[...], preferred_element_type=jnp.float32)
            + b1[...], 0.0)
        o_ref[...] = (jnp.dot(h1, w2[...], preferred_element_type=jnp.float32)
                      + b2[...])


def _fused_dense(x_all, agg, bids, rx_idx, wps, wpn, bp, wrs, wrn, br,
                 w1, b1, w2, b2):
    wspec = pl.BlockSpec((DIM, DIM), lambda i: (0, 0))
    bspec = pl.BlockSpec((1, DIM), lambda i: (0, 0))
    return pl.pallas_call(
        _fused_body,
        grid=(_NBLK,),
        in_specs=[
            pl.BlockSpec((_GNN_BLK, DIM), lambda i: (i, 0)),
            pl.BlockSpec((_GNN_BLK, DIM), lambda i: (i, 0)),
            pl.BlockSpec((_GNN_BLK, 1), lambda i: (i, 0)),
            pl.BlockSpec((1, R_SEG), lambda i: (0, 0)),
            wspec, wspec, bspec, wspec, wspec, bspec,
            pl.BlockSpec((2 * DIM, H_DIM), lambda i: (0, 0)),
            pl.BlockSpec((1, H_DIM), lambda i: (0, 0)),
            pl.BlockSpec((H_DIM, DIM), lambda i: (0, 0)),
            bspec,
        ],
        out_specs=pl.BlockSpec((T_SEG, DIM), lambda i: (0, 0)),
        out_shape=jax.ShapeDtypeStruct((T_SEG, DIM), jnp.float32),
        scratch_shapes=[pltpu.VMEM((_SEGS, DIM), jnp.float32)],
    )(x_all, agg, bids, rx_idx, wps, wpn, bp, wrs, wrn, br,
      w1, b1, w2, b2)


def kernel(x_prod, edge_index_prod, batch_prod, x_react, edge_index_react,
           batch_react, rxtant_indices, W_prod_self, W_prod_nbr, b_prod,
           W_react_self, W_react_nbr, b_react, W1, b1, W2, b2):
    x_all = jnp.concatenate(
        [x_prod, x_react, jnp.zeros((8, DIM), jnp.float32)], axis=0)
    npad = E_PAD - E_EDGES
    pad_iota = jnp.arange(npad, dtype=jnp.int32)
    src_fill = 2 * N_NODES + (pad_iota % 8)
    dst_fill = (pad_iota * 79) % N_NODES
    src_all = jnp.concatenate(
        [edge_index_prod[0], src_fill, edge_index_react[0] + N_NODES, src_fill])
    dst_all = jnp.concatenate(
        [edge_index_prod[1], dst_fill, edge_index_react[1], dst_fill])

    agg = _edge_agg(src_all, dst_all, x_all)

    bshift = jnp.concatenate(
        [batch_prod.astype(jnp.int32),
         batch_react.astype(jnp.int32) + T_SEG])
    return _fused_dense(x_all, agg, bshift.reshape(2 * N_NODES, 1),
                        rxtant_indices.reshape(1, R_SEG),
                        W_prod_self, W_prod_nbr, b_prod.reshape(1, DIM),
                        W_react_self, W_react_nbr, b_react.reshape(1, DIM),
                        W1, b1.reshape(1, H_DIM), W2, b2.reshape(1, DIM))


# speedup vs baseline: 1.3106x; 1.0291x over previous
import functools

import jax
import jax.numpy as jnp
from jax import lax
from jax.experimental import pallas as pl
from jax.experimental.pallas import tpu as pltpu
from jax.experimental.pallas import tpu_sc as plsc

NC, NS, L = 2, 16, 16

N_NODES = 10000
E_EDGES = 320000
DIM = 128
T_SEG = 512
R_SEG = 1024
H_DIM = 256

EDC = 128
EPT = 20480
NCHT = EPT // EDC
E_PAD = EPT * NS
ZR = 80
N_NCHUNK = N_NODES // ZR
POOL_C = 80
POOL_NCHUNK = N_NODES // POOL_C

_mesh = plsc.VectorSubcoreMesh(
    core_axis_name="c", subcore_axis_name="s", num_cores=NC, num_subcores=NS)


def _zero_vmem(buf, rows):
    def body(i, carry):
        r = i // (DIM // L)
        cc = (i % (DIM // L)) * L
        buf[r, pl.ds(cc, L)] = jnp.zeros((L,), jnp.float32)
        return carry
    lax.fori_loop(0, rows * (DIM // L), body, 0)


@functools.partial(
    pl.kernel,
    out_type=jax.ShapeDtypeStruct((2 * N_NODES, DIM), jnp.float32),
    mesh=_mesh,
    scratch_types=[
        pltpu.VMEM((EDC,), jnp.int32),
        pltpu.VMEM((EDC,), jnp.int32),
        pltpu.VMEM((EDC,), jnp.int32),
        pltpu.VMEM((EDC,), jnp.int32),
        pltpu.VMEM((EDC, DIM), jnp.float32),
        pltpu.VMEM((EDC, DIM), jnp.float32),
        pltpu.VMEM((ZR, DIM), jnp.float32),
        pltpu.VMEM_SHARED((N_NODES, DIM), jnp.float32),
        pltpu.SemaphoreType.DMA,
        pltpu.SemaphoreType.DMA,
        pltpu.SemaphoreType.DMA,
        pltpu.SemaphoreType.DMA,
    ],
)
def _edge_agg(src_all, dst_all, x_all, out, sbuf_a, sbuf_b, dbuf_a, dbuf_b,
              rows_a, rows_b, zbuf, acc, sem_a, sem_b, sem_ia, sem_ib):
    c = lax.axis_index("c")
    s = lax.axis_index("s")

    _zero_vmem(zbuf, ZR)

    def zbody(k, carry):
        g = s + k * NS

        @pl.when(g < N_NCHUNK)
        def _():
            pltpu.sync_copy(zbuf, acc.at[pl.ds(g * ZR, ZR)])
        return carry

    lax.fori_loop(0, (N_NCHUNK + NS - 1) // NS, zbody, 0)

    ebase = c * E_PAD + s * EPT

    pltpu.sync_copy(src_all.at[pl.ds(ebase, EDC)], sbuf_a)
    pltpu.sync_copy(dst_all.at[pl.ds(ebase, EDC)], dbuf_a)
    plsc.subcore_barrier()
    pltpu.async_copy(x_all.at[sbuf_a], rows_a, sem_a)
    pltpu.sync_copy(src_all.at[pl.ds(ebase + EDC, EDC)], sbuf_b)
    pltpu.sync_copy(dst_all.at[pl.ds(ebase + EDC, EDC)], dbuf_b)

    def body(k, carry):
        g0 = 2 * k
        more = g0 + 2 < NCHT

        pltpu.make_async_copy(x_all.at[sbuf_a], rows_a, sem_a).wait()
        pltpu.async_copy(x_all.at[sbuf_b], rows_b, sem_b)

        @pl.when(more)
        def _():
            base = ebase + (g0 + 2) * EDC
            pltpu.async_copy(src_all.at[pl.ds(base, EDC)], sbuf_a, sem_ia)

        pltpu.sync_copy(rows_a, acc.at[dbuf_a], add=True)

        @pl.when(more)
        def _():
            base = ebase + (g0 + 2) * EDC
            pltpu.async_copy(dst_all.at[pl.ds(base, EDC)], dbuf_a, sem_ia)

        pltpu.make_async_copy(x_all.at[sbuf_b], rows_b, sem_b).wait()

        @pl.when(more)
        def _():
            pltpu.make_async_copy(src_all.at[pl.ds(0, EDC)], sbuf_a, sem_ia).wait()
            pltpu.make_async_copy(dst_all.at[pl.ds(0, EDC)], dbuf_a, sem_ia).wait()
            pltpu.async_copy(x_all.at[sbuf_a], rows_a, sem_a)
            base = ebase + (g0 + 3) * EDC
            pltpu.async_copy(src_all.at[pl.ds(base, EDC)], sbuf_b, sem_ib)

        pltpu.sync_copy(rows_b, acc.at[dbuf_b], add=True)

        @pl.when(more)
        def _():
            base = ebase + (g0 + 3) * EDC
            pltpu.async_copy(dst_all.at[pl.ds(base, EDC)], dbuf_b, sem_ib)
            pltpu.make_async_copy(src_all.at[pl.ds(0, EDC)], sbuf_b, sem_ib).wait()
            pltpu.make_async_copy(dst_all.at[pl.ds(0, EDC)], dbuf_b, sem_ib).wait()
        return carry

    lax.fori_loop(0, NCHT // 2, body, 0)
    plsc.subcore_barrier()

    def obody(k, carry):
        g = s + k * NS

        @pl.when(g < N_NCHUNK)
        def _():
            pltpu.sync_copy(acc.at[pl.ds(g * ZR, ZR)],
                            out.at[pl.ds(c * N_NODES + g * ZR, ZR)])
        return carry

    lax.fori_loop(0, (N_NCHUNK + NS - 1) // NS, obody, 0)


_GNN_BLK = 1000


_NBLK = 2 * N_NODES // _GNN_BLK
_SEGS = T_SEG + R_SEG


def _self_body(x_ref, wps, wrs, bp, br, o_ref):
    i = pl.program_id(0)
    is_prod = i < (N_NODES // _GNN_BLK)
    ws = jnp.where(is_prod, wps[...], wrs[...])
    b = jnp.where(is_prod, bp[...], br[...])
    o_ref[...] = (jnp.dot(x_ref[...], ws, preferred_element_type=jnp.float32)
                  + b)


def _self_pre(x_all, wps, wrs, bp, br):
    wspec = pl.BlockSpec((DIM, DIM), lambda i: (0, 0))
    bspec = pl.BlockSpec((1, DIM), lambda i: (0, 0))
    return pl.pallas_call(
        _self_body,
        grid=(_NBLK,),
        in_specs=[pl.BlockSpec((_GNN_BLK, DIM), lambda i: (i, 0)),
                  wspec, wspec, bspec, bspec],
        out_specs=pl.BlockSpec((_GNN_BLK, DIM), lambda i: (i, 0)),
        out_shape=jax.ShapeDtypeStruct((2 * N_NODES, DIM), jnp.float32),
    )(x_all, wps, wrs, bp, br)


def _fused_body(s_ref, a_ref, bid_ref, ridx_ref, wpn, wrn, w1, b1, w2, b2,
                o_ref, acc_ref):
    i = pl.program_id(0)
    is_prod = i < (N_NODES // _GNN_BLK)
    wn = jnp.where(is_prod, wpn[...], wrn[...])
    h = jnp.maximum(
        s_ref[...]
        + jnp.dot(a_ref[...], wn, preferred_element_type=jnp.float32), 0.0)

    seg = lax.broadcasted_iota(jnp.int32, (_GNN_BLK, _SEGS), 1)
    onehot = (seg == bid_ref[...]).astype(jnp.float32)
    part = lax.dot_general(onehot, h, (((0,), (0,)), ((), ())),
                           preferred_element_type=jnp.float32)

    @pl.when(i == 0)
    def _():
        acc_ref[...] = part

    @pl.when(i > 0)
    def _():
        acc_ref[...] += part

    @pl.when(i == _NBLK - 1)
    def _():
        prods = acc_ref[0:T_SEG, :]
        rxt = jnp.maximum(acc_ref[T_SEG:, :], 0.0)
        ids2 = ridx_ref[...]
        t2 = lax.broadcasted_iota(jnp.int32, (T_SEG, R_SEG), 0)
        oh2 = (t2 == ids2).astype(jnp.float32)
        pooled = jnp.dot(oh2, rxt, preferred_element_type=jnp.float32)
        feats = jnp.concatenate([prods, pooled], axis=1)
        h1 = jnp.maximum(
            jnp.dot(feats, w1[...], preferred_element_type=jnp.float32)
            + b1[...], 0.0)
        o_ref[...] = (jnp.dot(h1, w2[...], preferred_element_type=jnp.float32)
                      + b2[...])


def _fused_dense(self_pre, agg, bids, rx_idx, wpn, wrn, w1, b1, w2, b2):
    wspec = pl.BlockSpec((DIM, DIM), lambda i: (0, 0))
    bspec = pl.BlockSpec((1, DIM), lambda i: (0, 0))
    return pl.pallas_call(
        _fused_body,
        grid=(_NBLK,),
        in_specs=[
            pl.BlockSpec((_GNN_BLK, DIM), lambda i: (i, 0)),
            pl.BlockSpec((_GNN_BLK, DIM), lambda i: (i, 0)),
            pl.BlockSpec((_GNN_BLK, 1), lambda i: (i, 0)),
            pl.BlockSpec((1, R_SEG), lambda i: (0, 0)),
            wspec, wspec,
            pl.BlockSpec((2 * DIM, H_DIM), lambda i: (0, 0)),
            pl.BlockSpec((1, H_DIM), lambda i: (0, 0)),
            pl.BlockSpec((H_DIM, DIM), lambda i: (0, 0)),
            bspec,
        ],
        out_specs=pl.BlockSpec((T_SEG, DIM), lambda i: (0, 0)),
        out_shape=jax.ShapeDtypeStruct((T_SEG, DIM), jnp.float32),
        scratch_shapes=[pltpu.VMEM((_SEGS, DIM), jnp.float32)],
    )(self_pre, agg, bids, rx_idx, wpn, wrn, w1, b1, w2, b2)


def kernel(x_prod, edge_index_prod, batch_prod, x_react, edge_index_react,
           batch_react, rxtant_indices, W_prod_self, W_prod_nbr, b_prod,
           W_react_self, W_react_nbr, b_react, W1, b1, W2, b2):
    x_all = jnp.concatenate(
        [x_prod, x_react, jnp.zeros((8, DIM), jnp.float32)], axis=0)
    npad = E_PAD - E_EDGES
    pad_iota = jnp.arange(npad, dtype=jnp.int32)
    src_fill = 2 * N_NODES + (pad_iota % 8)
    dst_fill = (pad_iota * 79) % N_NODES
    src_all = jnp.concatenate(
        [edge_index_prod[0], src_fill, edge_index_react[0] + N_NODES, src_fill])
    dst_all = jnp.concatenate(
        [edge_index_prod[1], dst_fill, edge_index_react[1], dst_fill])

    self_pre = _self_pre(x_all[: 2 * N_NODES], W_prod_self, W_react_self,
                         b_prod.reshape(1, DIM), b_react.reshape(1, DIM))
    agg = _edge_agg(src_all, dst_all, x_all)

    bshift = jnp.concatenate(
        [batch_prod.astype(jnp.int32),
         batch_react.astype(jnp.int32) + T_SEG])
    return _fused_dense(self_pre, agg, bshift.reshape(2 * N_NODES, 1),
                        rxtant_indices.reshape(1, R_SEG),
                        W_prod_nbr, W_react_nbr,
                        W1, b1.reshape(1, H_DIM), W2, b2.reshape(1, DIM))
